# baseline (speedup 1.0000x reference)
import functools

import jax
import jax.numpy as jnp
from jax import lax
from jax.experimental import pallas as pl
from jax.experimental.pallas import tpu as pltpu
from jax.experimental.pallas import tpu_sc as plsc

_NSUB = 16
_L = 16


def _sc_stage(props_pad, gt_boxes, table, n_rows, n_gt):
    per = props_pad.shape[0] // _NSUB
    chunks = per // _L
    tw = table.shape[1]

    def body(props_hbm, gt_hbm, table_hbm, rows_out, gtind_out,
             prop_v, gt_v, pmax_v, pidx_v, shared_max, shared_idx,
             allmax_v, allidx_v, gtind_v, rows_v, sem):
        sid = lax.axis_index("s")
        base = sid * per
        pltpu.sync_copy(props_hbm.at[pl.ds(base, per)], prop_v)
        pltpu.sync_copy(gt_hbm, gt_v)
        iota = lax.iota(jnp.int32, _L)
        col = [jnp.full((_L,), c, jnp.int32) for c in range(4)]

        gvals = []
        for cc in range((n_gt + _L - 1) // _L):
            rows = cc * _L + iota
            msk = rows < n_gt
            cy = plsc.load_gather(gt_v, [rows, col[0]], mask=msk)
            cx = plsc.load_gather(gt_v, [rows, col[1]], mask=msk)
            hh = plsc.load_gather(gt_v, [rows, col[2]], mask=msk)
            ww = plsc.load_gather(gt_v, [rows, col[3]], mask=msk)
            gy1 = cy - hh * 0.5
            gx1 = cx - ww * 0.5
            gy2 = cy + hh * 0.5
            gx2 = cx + ww * 0.5
            gab = (gy2 - gy1) * (gx2 - gx1)
            gvals.append((gy1, gx1, gy2, gx2, gab))

        pm = [jnp.zeros((_L,), jnp.float32) for _ in range(2)]
        pi = [jnp.zeros((_L,), jnp.int32) for _ in range(2)]
        neg_inf = jnp.float32(-jnp.inf)
        for j in range(n_gt):
            cc, l = divmod(j, _L)
            lane = iota == l

            def bc(v):
                return jnp.full((_L,), jnp.max(jnp.where(lane, v, neg_inf)),
                                jnp.float32)

            gy1b, gx1b, gy2b, gx2b, gabb = [bc(v) for v in gvals[cc]]

            def loop(c, carry):
                m, mi = carry
                rows = c * _L + iota
                py1 = plsc.load_gather(prop_v, [rows, col[0]])
                px1 = plsc.load_gather(prop_v, [rows, col[1]])
                py2 = plsc.load_gather(prop_v, [rows, col[2]])
                px2 = plsc.load_gather(prop_v, [rows, col[3]])
                aa = (py2 - py1) * (px2 - px1)
                iy1 = jnp.maximum(py1, gy1b)
                ix1 = jnp.maximum(px1, gx1b)
                iy2 = jnp.minimum(py2, gy2b)
                ix2 = jnp.minimum(px2, gx2b)
                ih = jnp.maximum(iy2 - iy1, 0.0)
                iw = jnp.maximum(ix2 - ix1, 0.0)
                inter = ih * iw
                iou = inter / (aa + gabb - inter)
                upd = iou > m
                gidx = base + rows
                return jnp.where(upd, iou, m), jnp.where(upd, gidx, mi)

            m, mi = lax.fori_loop(
                0, chunks, loop,
                (jnp.full((_L,), -1.0, jnp.float32),
                 jnp.zeros((_L,), jnp.int32)))
            mm = jnp.max(m)
            cand = jnp.where(m == mm, mi, jnp.int32(2**30))
            bi = jnp.min(cand)
            pm[cc] = jnp.where(lane, mm, pm[cc])
            pi[cc] = jnp.where(lane, bi, pi[cc])

        pmax_v[0, :] = pm[0]
        pmax_v[1, :] = pm[1]
        pidx_v[0, :] = pi[0]
        pidx_v[1, :] = pi[1]
        pltpu.sync_copy(pmax_v, shared_max.at[sid])
        pltpu.sync_copy(pidx_v, shared_idx.at[sid])
        plsc.subcore_barrier()

        @pl.when(sid == 0)
        def _():
            pltpu.sync_copy(shared_max, allmax_v)
            pltpu.sync_copy(shared_idx, allidx_v)
            for cc in range(2):
                cm = jnp.full((_L,), -2.0, jnp.float32)
                ci = jnp.zeros((_L,), jnp.int32)
                for w in range(_NSUB):
                    vm = allmax_v[w, cc, :]
                    vi = allidx_v[w, cc, :]
                    better = (vm > cm) | ((vm == cm) & (vi < ci))
                    cm = jnp.where(better, vm, cm)
                    ci = jnp.where(better, vi, ci)
                gtind_v[pl.ds(cc * _L, _L)] = ci
            pltpu.async_copy(table_hbm.at[gtind_v], rows_v, sem).wait()
            pltpu.sync_copy(rows_v, rows_out)
            pltpu.sync_copy(gtind_v, gtind_out)

    mesh = plsc.VectorSubcoreMesh(core_axis_name="c", subcore_axis_name="s",
                                  num_cores=1, num_subcores=_NSUB)
    f = pl.kernel(
        body, mesh=mesh,
        compiler_params=pltpu.CompilerParams(needs_layout_passes=False,
                                             use_tc_tiling_on_sc=False),
        out_type=(jax.ShapeDtypeStruct((2 * _L, tw), jnp.float32),
                  jax.ShapeDtypeStruct((2 * _L,), jnp.int32)),
        scratch_types=[
            pltpu.VMEM((per, 4), jnp.float32),
            pltpu.VMEM((n_gt, 4), jnp.float32),
            pltpu.VMEM((2, _L), jnp.float32),
            pltpu.VMEM((2, _L), jnp.int32),
            pltpu.VMEM_SHARED((_NSUB, 2, _L), jnp.float32),
            pltpu.VMEM_SHARED((_NSUB, 2, _L), jnp.int32),
            pltpu.VMEM((_NSUB, 2, _L), jnp.float32),
            pltpu.VMEM((_NSUB, 2, _L), jnp.int32),
            pltpu.VMEM((2 * _L,), jnp.int32),
            pltpu.VMEM((2 * _L, tw), jnp.float32),
            pltpu.SemaphoreType.DMA,
        ])
    return f(props_pad, gt_boxes, table)


def _tc_body(rows_ref, gtT_ref, gtind_ref, labels_ref, out_ref, *, n_gt, n_cls):
    rows = rows_ref[...]
    gtT = gtT_ref[...]
    gi = gtind_ref[...]
    lab = labels_ref[...]
    M, C = n_gt, n_cls

    p_y1 = rows[:, 25:26]
    p_x1 = rows[:, 26:27]
    p_y2 = rows[:, 27:28]
    p_x2 = rows[:, 28:29]
    r_cy = (p_y1 + p_y2) * 0.5
    r_cx = (p_x1 + p_x2) * 0.5
    r_h = p_y2 - p_y1
    r_w = p_x2 - p_x1
    gcy = gtT[0:1, :]
    gcx = gtT[1:2, :]
    gh = gtT[2:3, :]
    gw = gtT[3:4, :]
    gy1 = gcy - gh * 0.5
    gx1 = gcx - gw * 0.5
    gy2 = gcy + gh * 0.5
    gx2 = gcx + gw * 0.5
    area_b = (gy2 - gy1) * (gx2 - gx1)
    area_a = (p_y2 - p_y1) * (p_x2 - p_x1)
    iy1 = jnp.maximum(p_y1, gy1)
    ix1 = jnp.maximum(p_x1, gx1)
    iy2 = jnp.minimum(p_y2, gy2)
    ix2 = jnp.minimum(p_x2, gx2)
    ih = jnp.maximum(iy2 - iy1, 0.0)
    iw = jnp.maximum(ix2 - ix1, 0.0)
    inter = ih * iw
    iou = inter / (area_a + area_b - inter)

    iotaC = lax.broadcasted_iota(jnp.int32, (M, M), 1)
    iotaR = lax.broadcasted_iota(jnp.int32, (M, M), 0)
    mrow = jnp.max(iou, axis=1, keepdims=True)
    g = jnp.min(jnp.where(iou == mrow, iotaC, jnp.int32(2**30)),
                axis=1, keepdims=True)
    onehot = (iotaC == g).astype(jnp.float32)
    tcy = jnp.sum(onehot * gcy, axis=1, keepdims=True)
    tcx = jnp.sum(onehot * gcx, axis=1, keepdims=True)
    th = jnp.sum(onehot * gh, axis=1, keepdims=True)
    tw = jnp.sum(onehot * gw, axis=1, keepdims=True)
    ty = (tcy - r_cy) / r_h
    tx = (tcx - r_cx) / r_w
    tlh = jnp.log(th / r_h)
    tlw = jnp.log(tw / r_w)
    labsel = jnp.sum(onehot * lab, axis=1, keepdims=True)

    ident = (iotaR == iotaC).astype(jnp.float32)
    A = jnp.broadcast_to(gi, (M, M))
    gcol = jnp.sum(A * ident, axis=1, keepdims=True)
    dup = (A == gcol) & (iotaC < iotaR)
    isf = jnp.sum(dup.astype(jnp.float32), axis=1, keepdims=True) == 0.0
    cnt = jnp.sum(isf.astype(jnp.float32))

    rb_y = rows[:, 0:1]
    rb_x = rows[:, 1:2]
    rb_h = rows[:, 2:3]
    rb_w = rows[:, 3:4]
    lx = jnp.sum(jnp.where(isf, jnp.abs(rb_x - tx), 0.0)) / cnt
    ly = jnp.sum(jnp.where(isf, jnp.abs(rb_y - ty), 0.0)) / cnt
    lh = jnp.sum(jnp.where(isf, jnp.abs(rb_h - tlh), 0.0)) / cnt
    lw = jnp.sum(jnp.where(isf, jnp.abs(rb_w - tlw), 0.0)) / cnt

    rc = rows[:, 4:4 + C]
    rmax = jnp.max(rc, axis=1, keepdims=True)
    lse = jnp.log(jnp.sum(jnp.exp(rc - rmax), axis=1, keepdims=True)) + rmax
    iotaCls = lax.broadcasted_iota(jnp.int32, (M, C), 1).astype(jnp.float32)
    picked = jnp.sum(jnp.where(iotaCls == labsel, rc, 0.0),
                     axis=1, keepdims=True)
    lc = jnp.sum(jnp.where(isf, lse - picked, 0.0)) / cnt

    il = lax.broadcasted_iota(jnp.int32, (1, 128), 1)
    acc = jnp.where(il == 0, lx, 0.0)
    acc = jnp.where(il == 1, ly, acc)
    acc = jnp.where(il == 2, lh, acc)
    acc = jnp.where(il == 3, lw, acc)
    acc = jnp.where(il == 4, lc, acc)
    out_ref[...] = acc


def _tc_stage(rows, gtT, gtind_f, labels_f, n_gt, n_cls, interpret=False):
    return pl.pallas_call(
        functools.partial(_tc_body, n_gt=n_gt, n_cls=n_cls),
        out_shape=jax.ShapeDtypeStruct((1, 128), jnp.float32),
        interpret=interpret,
    )(rows, gtT, gtind_f, labels_f)


def kernel(proposal_rois, rcnn_boxes, rcnn_class, gt_boxes, gt_labels):
    N = proposal_rois.shape[0]
    M = gt_boxes.shape[0]
    C = rcnn_class.shape[-1]
    per = -(-N // (_NSUB * _L)) * _L
    props_pad = jnp.pad(proposal_rois, ((0, _NSUB * per - N), (0, 0)))
    table = jnp.concatenate(
        [rcnn_boxes[0], rcnn_class[0], proposal_rois,
         jnp.zeros((N, 32 - 8 - C), jnp.float32)], axis=1)
    rows, gtind = _sc_stage(props_pad, gt_boxes, table, N, M)
    gtT = gt_boxes.T
    gtind_f = gtind[:M].astype(jnp.float32).reshape(1, M)
    labels_f = gt_labels[:M].astype(jnp.float32).reshape(1, M)
    out = _tc_stage(rows[:M], gtT, gtind_f, labels_f, M, C)
    return tuple(out[0, k].reshape(()) for k in range(5))
